# initial kernel scaffold (unmeasured)
import jax
import jax.numpy as jnp
from jax import lax
from jax.experimental import pallas as pl
from jax.experimental.pallas import tpu as pltpu

F32 = jnp.float32

B, S, D = 4, 256, 4096
H, Dh, Dr = 32, 128, 64
DC = 128


def _exchange(x2, Wdkv, Wuk, Wuv):
    M = x2.shape[0]
    N = Wuk.shape[1]

    def body(x_ref, wdkv_ref, wuk_ref, wuv_ref,
             cmine_ref, cother_ref, wuko_ref, wuvo_ref,
             send_sems, recv_sems):
        my_x = lax.axis_index("x")
        my_y = lax.axis_index("y")
        my_z = lax.axis_index("z")
        partner = (1 - my_x, my_y, my_z)

        barrier = pltpu.get_barrier_semaphore()
        pl.semaphore_signal(barrier, inc=1, device_id=partner,
                            device_id_type=pl.DeviceIdType.MESH)
        pl.semaphore_wait(barrier, 1)

        rdma_wuk = pltpu.make_async_remote_copy(
            src_ref=wuk_ref, dst_ref=wuko_ref,
            send_sem=send_sems.at[0], recv_sem=recv_sems.at[0],
            device_id=partner, device_id_type=pl.DeviceIdType.MESH)
        rdma_wuk.start()
        rdma_wuv = pltpu.make_async_remote_copy(
            src_ref=wuv_ref, dst_ref=wuvo_ref,
            send_sem=send_sems.at[1], recv_sem=recv_sems.at[1],
            device_id=partner, device_id_type=pl.DeviceIdType.MESH)
        rdma_wuv.start()

        cmine_ref[...] = jnp.dot(x_ref[...], wdkv_ref[...],
                                 preferred_element_type=F32)

        rdma_c = pltpu.make_async_remote_copy(
            src_ref=cmine_ref, dst_ref=cother_ref,
            send_sem=send_sems.at[2], recv_sem=recv_sems.at[2],
            device_id=partner, device_id_type=pl.DeviceIdType.MESH)
        rdma_c.start()

        rdma_wuk.wait()
        rdma_wuv.wait()
        rdma_c.wait()

    return pl.pallas_call(
        body,
        out_shape=[
            jax.ShapeDtypeStruct((M, DC), F32),
            jax.ShapeDtypeStruct((M, DC), F32),
            jax.ShapeDtypeStruct((DC, N), F32),
            jax.ShapeDtypeStruct((DC, N), F32),
        ],
        in_specs=[pl.BlockSpec(memory_space=pltpu.VMEM)] * 4,
        out_specs=[pl.BlockSpec(memory_space=pltpu.VMEM)] * 4,
        scratch_shapes=[
            pltpu.SemaphoreType.DMA((3,)),
            pltpu.SemaphoreType.DMA((3,)),
        ],
        compiler_params=pltpu.CompilerParams(collective_id=0),
    )(x2, Wdkv, Wuk, Wuv)


def _matmul(a, b, block_n=512):
    M, K = a.shape
    _, N = b.shape
    block_n = min(block_n, N)

    def body(a_ref, b_ref, o_ref):
        o_ref[...] = jnp.dot(a_ref[...], b_ref[...],
                             preferred_element_type=F32)

    return pl.pallas_call(
        body,
        grid=(N // block_n,),
        in_specs=[
            pl.BlockSpec((M, K), lambda j: (0, 0)),
            pl.BlockSpec((K, block_n), lambda j: (0, j)),
        ],
        out_specs=pl.BlockSpec((M, block_n), lambda j: (0, j)),
        out_shape=jax.ShapeDtypeStruct((M, N), F32),
    )(a, b)


def _matmul2(a1, b1, a2, b2, block_n=512):
    M, K = a1.shape
    _, N = b1.shape
    block_n = min(block_n, N)

    def body(a1_ref, b1_ref, a2_ref, b2_ref, o_ref):
        o_ref[...] = (
            jnp.dot(a1_ref[...], b1_ref[...], preferred_element_type=F32)
            + jnp.dot(a2_ref[...], b2_ref[...], preferred_element_type=F32)
        )

    return pl.pallas_call(
        body,
        grid=(N // block_n,),
        in_specs=[
            pl.BlockSpec((M, K), lambda j: (0, 0)),
            pl.BlockSpec((K, block_n), lambda j: (0, j)),
            pl.BlockSpec((M, K), lambda j: (0, 0)),
            pl.BlockSpec((K, block_n), lambda j: (0, j)),
        ],
        out_specs=pl.BlockSpec((M, block_n), lambda j: (0, j)),
        out_shape=jax.ShapeDtypeStruct((M, N), F32),
    )(a1, b1, a2, b2)


def _attention(Q, K, V, Qr, Kr):
    scale = (Dh + Dr) ** -0.5

    def body(q_ref, k_ref, qr_ref, kr_ref, v_ref, o_ref):
        s = lax.dot_general(q_ref[...], k_ref[...],
                            (((1,), (1,)), ((), ())),
                            preferred_element_type=F32)
        sr = lax.dot_general(qr_ref[...], kr_ref[...],
                             (((1,), (1,)), ((), ())),
                             preferred_element_type=F32)
        s = (s + sr) * scale
        m = jnp.max(s, axis=-1, keepdims=True)
        p = jnp.exp(s - m)
        p = p / jnp.sum(p, axis=-1, keepdims=True)
        o_ref[...] = jnp.dot(p, v_ref[...], preferred_element_type=F32)

    return pl.pallas_call(
        body,
        grid=(B, H),
        in_specs=[
            pl.BlockSpec((S, Dh), lambda b, h: (b, h)),
            pl.BlockSpec((S, Dh), lambda b, h: (b, h)),
            pl.BlockSpec((S, Dr), lambda b, h: (b, h)),
            pl.BlockSpec((S, Dr), lambda b, h: (b, 0)),
            pl.BlockSpec((S, Dh), lambda b, h: (b, h)),
        ],
        out_specs=pl.BlockSpec((S, Dh), lambda b, h: (b, h)),
        out_shape=jax.ShapeDtypeStruct((B * S, H * Dh), F32),
    )(Q, K, Qr, Kr, V)


def kernel(x, Wdkv, Wuk, Wuv, Wq, Wqr, Wkr, Wo):
    x2 = x.reshape(B * S, D)
    c_mine, c_other, wuk_o, wuv_o = _exchange(x2, Wdkv, Wuk, Wuv)
    K = _matmul2(c_mine, Wuk, c_other, wuk_o)
    V = _matmul2(c_mine, Wuv, c_other, wuv_o)
    Q = _matmul(x2, Wq)
    Qr = _matmul(x2, Wqr)
    Kr = _matmul(x2, Wkr)
    O = _attention(Q, K, V, Qr, Kr)
    out = _matmul(O, Wo)
    return out.reshape(B, S, D)


# baseline (device time: 296397 ns/iter reference)
import jax
import jax.numpy as jnp
from jax import lax
from jax.experimental import pallas as pl
from jax.experimental.pallas import tpu as pltpu

F32 = jnp.float32

B, S, D = 4, 256, 4096
H, Dh, Dr = 32, 128, 64
DC = 128


def _exchange(x2, Wdkv, Wuk, Wuv):
    M = x2.shape[0]
    N = Wuk.shape[1]

    def body(x_ref, wdkv_ref, wuk_ref, wuv_ref,
             cmine_ref, cother_ref, wuko_ref, wuvo_ref,
             send_sems, recv_sems):
        my_x = lax.axis_index("x")
        my_y = lax.axis_index("y")
        my_z = lax.axis_index("z")
        partner = (1 - my_x, my_y, my_z)

        barrier = pltpu.get_barrier_semaphore()
        pl.semaphore_signal(barrier, inc=1, device_id=partner,
                            device_id_type=pl.DeviceIdType.MESH)
        pl.semaphore_wait(barrier, 1)

        rdma_wuk = pltpu.make_async_remote_copy(
            src_ref=wuk_ref, dst_ref=wuko_ref,
            send_sem=send_sems.at[0], recv_sem=recv_sems.at[0],
            device_id=partner, device_id_type=pl.DeviceIdType.MESH)
        rdma_wuk.start()
        rdma_wuv = pltpu.make_async_remote_copy(
            src_ref=wuv_ref, dst_ref=wuvo_ref,
            send_sem=send_sems.at[1], recv_sem=recv_sems.at[1],
            device_id=partner, device_id_type=pl.DeviceIdType.MESH)
        rdma_wuv.start()

        cmine_ref[...] = jnp.dot(x_ref[...], wdkv_ref[...],
                                 preferred_element_type=F32)

        rdma_c = pltpu.make_async_remote_copy(
            src_ref=cmine_ref, dst_ref=cother_ref,
            send_sem=send_sems.at[2], recv_sem=recv_sems.at[2],
            device_id=partner, device_id_type=pl.DeviceIdType.MESH)
        rdma_c.start()

        rdma_wuk.wait()
        rdma_wuv.wait()
        rdma_c.wait()

    return pl.pallas_call(
        body,
        out_shape=[
            jax.ShapeDtypeStruct((M, DC), F32),
            jax.ShapeDtypeStruct((M, DC), F32),
            jax.ShapeDtypeStruct((DC, N), F32),
            jax.ShapeDtypeStruct((DC, N), F32),
        ],
        in_specs=[pl.BlockSpec(memory_space=pltpu.VMEM)] * 4,
        out_specs=[pl.BlockSpec(memory_space=pltpu.VMEM)] * 4,
        scratch_shapes=[
            pltpu.SemaphoreType.DMA((3,)),
            pltpu.SemaphoreType.DMA((3,)),
        ],
        compiler_params=pltpu.CompilerParams(collective_id=0),
    )(x2, Wdkv, Wuk, Wuv)


def _matmul(a, b, block_n=256):
    M, K = a.shape
    _, N = b.shape
    block_n = min(block_n, N)

    def body(a_ref, b_ref, o_ref):
        o_ref[...] = jnp.dot(a_ref[...], b_ref[...],
                             preferred_element_type=F32)

    return pl.pallas_call(
        body,
        grid=(N // block_n,),
        in_specs=[
            pl.BlockSpec((M, K), lambda j: (0, 0)),
            pl.BlockSpec((K, block_n), lambda j: (0, j)),
        ],
        out_specs=pl.BlockSpec((M, block_n), lambda j: (0, j)),
        out_shape=jax.ShapeDtypeStruct((M, N), F32),
    )(a, b)


def _matmul2(a1, b1, a2, b2, block_n=512):
    M, K = a1.shape
    _, N = b1.shape
    block_n = min(block_n, N)

    def body(a1_ref, b1_ref, a2_ref, b2_ref, o_ref):
        o_ref[...] = (
            jnp.dot(a1_ref[...], b1_ref[...], preferred_element_type=F32)
            + jnp.dot(a2_ref[...], b2_ref[...], preferred_element_type=F32)
        )

    return pl.pallas_call(
        body,
        grid=(N // block_n,),
        in_specs=[
            pl.BlockSpec((M, K), lambda j: (0, 0)),
            pl.BlockSpec((K, block_n), lambda j: (0, j)),
            pl.BlockSpec((M, K), lambda j: (0, 0)),
            pl.BlockSpec((K, block_n), lambda j: (0, j)),
        ],
        out_specs=pl.BlockSpec((M, block_n), lambda j: (0, j)),
        out_shape=jax.ShapeDtypeStruct((M, N), F32),
    )(a1, b1, a2, b2)


def _attention(Q, K, V, Qr, Kr):
    scale = (Dh + Dr) ** -0.5

    HB = 8

    def body(q_ref, k_ref, qr_ref, kr_ref, v_ref, o_ref):
        kr = kr_ref[...]
        for h in range(HB):
            q = q_ref[:, h * Dh:(h + 1) * Dh]
            k = k_ref[:, h * Dh:(h + 1) * Dh]
            qr = qr_ref[:, h * Dr:(h + 1) * Dr]
            s = lax.dot_general(q, k, (((1,), (1,)), ((), ())),
                                preferred_element_type=F32)
            sr = lax.dot_general(qr, kr, (((1,), (1,)), ((), ())),
                                 preferred_element_type=F32)
            s = (s + sr) * scale
            m = jnp.max(s, axis=-1, keepdims=True)
            p = jnp.exp(s - m)
            p = p / jnp.sum(p, axis=-1, keepdims=True)
            o_ref[:, h * Dh:(h + 1) * Dh] = jnp.dot(
                p, v_ref[:, h * Dh:(h + 1) * Dh], preferred_element_type=F32)

    return pl.pallas_call(
        body,
        grid=(B, H // HB),
        in_specs=[
            pl.BlockSpec((S, HB * Dh), lambda b, g: (b, g)),
            pl.BlockSpec((S, HB * Dh), lambda b, g: (b, g)),
            pl.BlockSpec((S, HB * Dr), lambda b, g: (b, g)),
            pl.BlockSpec((S, Dr), lambda b, g: (b, 0)),
            pl.BlockSpec((S, HB * Dh), lambda b, g: (b, g)),
        ],
        out_specs=pl.BlockSpec((S, HB * Dh), lambda b, g: (b, g)),
        out_shape=jax.ShapeDtypeStruct((B * S, H * Dh), F32),
    )(Q, K, Qr, Kr, V)


def kernel(x, Wdkv, Wuk, Wuv, Wq, Wqr, Wkr, Wo):
    x2 = x.reshape(B * S, D)
    c_mine, c_other, wuk_o, wuv_o = _exchange(x2, Wdkv, Wuk, Wuv)
    K = _matmul2(c_mine, Wuk, c_other, wuk_o)
    V = _matmul2(c_mine, Wuv, c_other, wuv_o)
    Q = _matmul(x2, Wq)
    Qr = _matmul(x2, Wqr)
    Kr = _matmul(x2, Wkr)
    O = _attention(Q, K, V, Qr, Kr)
    out = _matmul(O, Wo)
    return out.reshape(B, S, D)


# device time: 256923 ns/iter; 1.1536x vs baseline; 1.1536x over previous
import jax
import jax.numpy as jnp
from jax import lax
from jax.experimental import pallas as pl
from jax.experimental.pallas import tpu as pltpu

F32 = jnp.float32

B, S, D = 4, 256, 4096
H, Dh, Dr = 32, 128, 64
DC = 128


def _exchange_q(x2, Wdkv, Wuk, Wuv, Wq, block_n=256):
    M = x2.shape[0]
    N = Wq.shape[1]
    NW = Wuk.shape[1]
    nsteps = N // block_n

    def body(x_ref, wdkv_ref, wuk_ref, wuv_ref, wq_ref,
             q_ref, cmine_ref, cother_ref, wuko_ref, wuvo_ref,
             send_sems, recv_sems):
        j = pl.program_id(0)
        my_x = lax.axis_index("x")
        my_y = lax.axis_index("y")
        my_z = lax.axis_index("z")
        partner = (1 - my_x, my_y, my_z)

        def mk(src, dst, i):
            return pltpu.make_async_remote_copy(
                src_ref=src, dst_ref=dst,
                send_sem=send_sems.at[i], recv_sem=recv_sems.at[i],
                device_id=partner, device_id_type=pl.DeviceIdType.MESH)

        @pl.when(j == 0)
        def _():
            barrier = pltpu.get_barrier_semaphore()
            pl.semaphore_signal(barrier, inc=1, device_id=partner,
                                device_id_type=pl.DeviceIdType.MESH)
            pl.semaphore_wait(barrier, 1)
            mk(wuk_ref, wuko_ref, 0).start()
            mk(wuv_ref, wuvo_ref, 1).start()
            cmine_ref[...] = jnp.dot(x_ref[...], wdkv_ref[...],
                                     preferred_element_type=F32)
            mk(cmine_ref, cother_ref, 2).start()

        q_ref[...] = jnp.dot(x_ref[...], wq_ref[...],
                             preferred_element_type=F32)

        @pl.when(j == nsteps - 1)
        def _():
            mk(wuk_ref, wuko_ref, 0).wait()
            mk(wuv_ref, wuvo_ref, 1).wait()
            mk(cmine_ref, cother_ref, 2).wait()

    return pl.pallas_call(
        body,
        grid=(nsteps,),
        in_specs=[
            pl.BlockSpec(memory_space=pltpu.VMEM),
            pl.BlockSpec(memory_space=pltpu.VMEM),
            pl.BlockSpec(memory_space=pltpu.VMEM),
            pl.BlockSpec(memory_space=pltpu.VMEM),
            pl.BlockSpec((D, block_n), lambda j: (0, j)),
        ],
        out_specs=[
            pl.BlockSpec((M, block_n), lambda j: (0, j)),
            pl.BlockSpec(memory_space=pltpu.VMEM),
            pl.BlockSpec(memory_space=pltpu.VMEM),
            pl.BlockSpec(memory_space=pltpu.VMEM),
            pl.BlockSpec(memory_space=pltpu.VMEM),
        ],
        out_shape=[
            jax.ShapeDtypeStruct((M, N), F32),
            jax.ShapeDtypeStruct((M, DC), F32),
            jax.ShapeDtypeStruct((M, DC), F32),
            jax.ShapeDtypeStruct((DC, NW), F32),
            jax.ShapeDtypeStruct((DC, NW), F32),
        ],
        scratch_shapes=[
            pltpu.SemaphoreType.DMA((3,)),
            pltpu.SemaphoreType.DMA((3,)),
        ],
        compiler_params=pltpu.CompilerParams(
            collective_id=0,
            vmem_limit_bytes=100 * 1024 * 1024,
        ),
    )(x2, Wdkv, Wuk, Wuv, Wq)


def _matmul(a, b, block_n=256):
    M, K = a.shape
    _, N = b.shape
    block_n = min(block_n, N)

    def body(a_ref, b_ref, o_ref):
        o_ref[...] = jnp.dot(a_ref[...], b_ref[...],
                             preferred_element_type=F32)

    return pl.pallas_call(
        body,
        grid=(N // block_n,),
        in_specs=[
            pl.BlockSpec((M, K), lambda j: (0, 0)),
            pl.BlockSpec((K, block_n), lambda j: (0, j)),
        ],
        out_specs=pl.BlockSpec((M, block_n), lambda j: (0, j)),
        out_shape=jax.ShapeDtypeStruct((M, N), F32),
    )(a, b)


def _matmul2(a1, b1, a2, b2, block_n=512):
    M, K = a1.shape
    _, N = b1.shape
    block_n = min(block_n, N)

    def body(a1_ref, b1_ref, a2_ref, b2_ref, o_ref):
        o_ref[...] = (
            jnp.dot(a1_ref[...], b1_ref[...], preferred_element_type=F32)
            + jnp.dot(a2_ref[...], b2_ref[...], preferred_element_type=F32)
        )

    return pl.pallas_call(
        body,
        grid=(N // block_n,),
        in_specs=[
            pl.BlockSpec((M, K), lambda j: (0, 0)),
            pl.BlockSpec((K, block_n), lambda j: (0, j)),
            pl.BlockSpec((M, K), lambda j: (0, 0)),
            pl.BlockSpec((K, block_n), lambda j: (0, j)),
        ],
        out_specs=pl.BlockSpec((M, block_n), lambda j: (0, j)),
        out_shape=jax.ShapeDtypeStruct((M, N), F32),
    )(a1, b1, a2, b2)


def _attention(Q, K, V, Qr, Kr):
    scale = (Dh + Dr) ** -0.5

    HB = 8

    def body(q_ref, k_ref, qr_ref, kr_ref, v_ref, o_ref):
        kr = kr_ref[...]
        for h in range(HB):
            q = q_ref[:, h * Dh:(h + 1) * Dh]
            k = k_ref[:, h * Dh:(h + 1) * Dh]
            qr = qr_ref[:, h * Dr:(h + 1) * Dr]
            s = lax.dot_general(q, k, (((1,), (1,)), ((), ())),
                                preferred_element_type=F32)
            sr = lax.dot_general(qr, kr, (((1,), (1,)), ((), ())),
                                 preferred_element_type=F32)
            s = (s + sr) * scale
            m = jnp.max(s, axis=-1, keepdims=True)
            p = jnp.exp(s - m)
            p = p / jnp.sum(p, axis=-1, keepdims=True)
            o_ref[:, h * Dh:(h + 1) * Dh] = jnp.dot(
                p, v_ref[:, h * Dh:(h + 1) * Dh], preferred_element_type=F32)

    return pl.pallas_call(
        body,
        grid=(B, H // HB),
        in_specs=[
            pl.BlockSpec((S, HB * Dh), lambda b, g: (b, g)),
            pl.BlockSpec((S, HB * Dh), lambda b, g: (b, g)),
            pl.BlockSpec((S, HB * Dr), lambda b, g: (b, g)),
            pl.BlockSpec((S, Dr), lambda b, g: (b, 0)),
            pl.BlockSpec((S, HB * Dh), lambda b, g: (b, g)),
        ],
        out_specs=pl.BlockSpec((S, HB * Dh), lambda b, g: (b, g)),
        out_shape=jax.ShapeDtypeStruct((B * S, H * Dh), F32),
    )(Q, K, Qr, Kr, V)


def kernel(x, Wdkv, Wuk, Wuv, Wq, Wqr, Wkr, Wo):
    x2 = x.reshape(B * S, D)
    Q, c_mine, c_other, wuk_o, wuv_o = _exchange_q(x2, Wdkv, Wuk, Wuv, Wq)
    K = _matmul2(c_mine, Wuk, c_other, wuk_o)
    V = _matmul2(c_mine, Wuv, c_other, wuv_o)
    Qr = _matmul(x2, Wqr)
    Kr = _matmul(x2, Wkr)
    O = _attention(Q, K, V, Qr, Kr)
    out = _matmul(O, Wo)
    return out.reshape(B, S, D)


# device time: 231487 ns/iter; 1.2804x vs baseline; 1.1099x over previous
import jax
import jax.numpy as jnp
from jax import lax
from jax.experimental import pallas as pl
from jax.experimental.pallas import tpu as pltpu

F32 = jnp.float32

B, S, D = 4, 256, 4096
H, Dh, Dr = 32, 128, 64
DC = 128


def _proj_exchange(x2, Wdkv, Wuk, Wuv, Wq, Wqr, Wkr, block_n=256):
    M = x2.shape[0]
    N = Wq.shape[1]
    NR = Wqr.shape[1]
    NW = Wuk.shape[1]
    nsteps = N // block_n
    nr_steps = NR // block_n

    def body(x_ref, wdkv_ref, wuk_ref, wuv_ref, wq_ref, wqr_ref, wkr_ref,
             q_ref, qr_ref, kr_ref,
             cmine_ref, cother_ref, wuko_ref, wuvo_ref,
             send_sems, recv_sems):
        j = pl.program_id(0)
        my_x = lax.axis_index("x")
        my_y = lax.axis_index("y")
        my_z = lax.axis_index("z")
        partner = (1 - my_x, my_y, my_z)

        def mk(src, dst, i):
            return pltpu.make_async_remote_copy(
                src_ref=src, dst_ref=dst,
                send_sem=send_sems.at[i], recv_sem=recv_sems.at[i],
                device_id=partner, device_id_type=pl.DeviceIdType.MESH)

        @pl.when(j == 0)
        def _():
            barrier = pltpu.get_barrier_semaphore()
            pl.semaphore_signal(barrier, inc=1, device_id=partner,
                                device_id_type=pl.DeviceIdType.MESH)
            pl.semaphore_wait(barrier, 1)
            mk(wuk_ref, wuko_ref, 0).start()
            mk(wuv_ref, wuvo_ref, 1).start()
            cmine_ref[...] = jnp.dot(x_ref[...], wdkv_ref[...],
                                     preferred_element_type=F32)
            mk(cmine_ref, cother_ref, 2).start()
            kr_ref[...] = jnp.dot(x_ref[...], wkr_ref[...],
                                  preferred_element_type=F32)

        q_ref[...] = jnp.dot(x_ref[...], wq_ref[...],
                             preferred_element_type=F32)

        @pl.when(j < nr_steps)
        def _():
            qr_ref[...] = jnp.dot(x_ref[...], wqr_ref[...],
                                  preferred_element_type=F32)

        @pl.when(j == nsteps - 1)
        def _():
            mk(wuk_ref, wuko_ref, 0).wait()
            mk(wuv_ref, wuvo_ref, 1).wait()
            mk(cmine_ref, cother_ref, 2).wait()

    qr_idx = lambda j: (0, jnp.minimum(j, nr_steps - 1))
    return pl.pallas_call(
        body,
        grid=(nsteps,),
        in_specs=[
            pl.BlockSpec(memory_space=pltpu.VMEM),
            pl.BlockSpec(memory_space=pltpu.VMEM),
            pl.BlockSpec(memory_space=pltpu.VMEM),
            pl.BlockSpec(memory_space=pltpu.VMEM),
            pl.BlockSpec((D, block_n), lambda j: (0, j)),
            pl.BlockSpec((D, block_n), qr_idx),
            pl.BlockSpec(memory_space=pltpu.VMEM),
        ],
        out_specs=[
            pl.BlockSpec((M, block_n), lambda j: (0, j)),
            pl.BlockSpec((M, block_n), qr_idx),
            pl.BlockSpec(memory_space=pltpu.VMEM),
            pl.BlockSpec(memory_space=pltpu.VMEM),
            pl.BlockSpec(memory_space=pltpu.VMEM),
            pl.BlockSpec(memory_space=pltpu.VMEM),
            pl.BlockSpec(memory_space=pltpu.VMEM),
        ],
        out_shape=[
            jax.ShapeDtypeStruct((M, N), F32),
            jax.ShapeDtypeStruct((M, NR), F32),
            jax.ShapeDtypeStruct((M, Dr), F32),
            jax.ShapeDtypeStruct((M, DC), F32),
            jax.ShapeDtypeStruct((M, DC), F32),
            jax.ShapeDtypeStruct((DC, NW), F32),
            jax.ShapeDtypeStruct((DC, NW), F32),
        ],
        scratch_shapes=[
            pltpu.SemaphoreType.DMA((3,)),
            pltpu.SemaphoreType.DMA((3,)),
        ],
        compiler_params=pltpu.CompilerParams(
            collective_id=0,
            vmem_limit_bytes=100 * 1024 * 1024,
        ),
    )(x2, Wdkv, Wuk, Wuv, Wq, Wqr, Wkr)


def _matmul(a, b, block_n=256):
    M, K = a.shape
    _, N = b.shape
    block_n = min(block_n, N)

    def body(a_ref, b_ref, o_ref):
        o_ref[...] = jnp.dot(a_ref[...], b_ref[...],
                             preferred_element_type=F32)

    return pl.pallas_call(
        body,
        grid=(N // block_n,),
        in_specs=[
            pl.BlockSpec((M, K), lambda j: (0, 0)),
            pl.BlockSpec((K, block_n), lambda j: (0, j)),
        ],
        out_specs=pl.BlockSpec((M, block_n), lambda j: (0, j)),
        out_shape=jax.ShapeDtypeStruct((M, N), F32),
    )(a, b)


def _matmul2(a1, b1, a2, b2, block_n=512):
    M, K = a1.shape
    _, N = b1.shape
    block_n = min(block_n, N)

    def body(a1_ref, b1_ref, a2_ref, b2_ref, o_ref):
        o_ref[...] = (
            jnp.dot(a1_ref[...], b1_ref[...], preferred_element_type=F32)
            + jnp.dot(a2_ref[...], b2_ref[...], preferred_element_type=F32)
        )

    return pl.pallas_call(
        body,
        grid=(N // block_n,),
        in_specs=[
            pl.BlockSpec((M, K), lambda j: (0, 0)),
            pl.BlockSpec((K, block_n), lambda j: (0, j)),
            pl.BlockSpec((M, K), lambda j: (0, 0)),
            pl.BlockSpec((K, block_n), lambda j: (0, j)),
        ],
        out_specs=pl.BlockSpec((M, block_n), lambda j: (0, j)),
        out_shape=jax.ShapeDtypeStruct((M, N), F32),
    )(a1, b1, a2, b2)


def _attention(Q, Qr, Kr, c_mine, wuk_m, c_other, wuk_o, wuv_m, wuv_o):
    scale = (Dh + Dr) ** -0.5

    HB = 8

    def body(q_ref, qr_ref, kr_ref, cm_ref, co_ref,
             wukm_ref, wuko_ref, wuvm_ref, wuvo_ref, o_ref):
        cm = cm_ref[...]
        co = co_ref[...]
        k_blk = (jnp.dot(cm, wukm_ref[...], preferred_element_type=F32)
                 + jnp.dot(co, wuko_ref[...], preferred_element_type=F32))
        v_blk = (jnp.dot(cm, wuvm_ref[...], preferred_element_type=F32)
                 + jnp.dot(co, wuvo_ref[...], preferred_element_type=F32))
        kr = kr_ref[...]
        for h in range(HB):
            q = q_ref[:, h * Dh:(h + 1) * Dh]
            k = k_blk[:, h * Dh:(h + 1) * Dh]
            qr = qr_ref[:, h * Dr:(h + 1) * Dr]
            s = lax.dot_general(q, k, (((1,), (1,)), ((), ())),
                                preferred_element_type=F32)
            sr = lax.dot_general(qr, kr, (((1,), (1,)), ((), ())),
                                 preferred_element_type=F32)
            s = (s + sr) * scale
            m = jnp.max(s, axis=-1, keepdims=True)
            p = jnp.exp(s - m)
            p = p / jnp.sum(p, axis=-1, keepdims=True)
            o_ref[:, h * Dh:(h + 1) * Dh] = jnp.dot(
                p, v_blk[:, h * Dh:(h + 1) * Dh], preferred_element_type=F32)

    return pl.pallas_call(
        body,
        grid=(B, H // HB),
        in_specs=[
            pl.BlockSpec((S, HB * Dh), lambda b, g: (b, g)),
            pl.BlockSpec((S, HB * Dr), lambda b, g: (b, g)),
            pl.BlockSpec((S, Dr), lambda b, g: (b, 0)),
            pl.BlockSpec((S, DC), lambda b, g: (b, 0)),
            pl.BlockSpec((S, DC), lambda b, g: (b, 0)),
            pl.BlockSpec((DC, HB * Dh), lambda b, g: (0, g)),
            pl.BlockSpec((DC, HB * Dh), lambda b, g: (0, g)),
            pl.BlockSpec((DC, HB * Dh), lambda b, g: (0, g)),
            pl.BlockSpec((DC, HB * Dh), lambda b, g: (0, g)),
        ],
        out_specs=pl.BlockSpec((S, HB * Dh), lambda b, g: (b, g)),
        out_shape=jax.ShapeDtypeStruct((B * S, H * Dh), F32),
    )(Q, Qr, Kr, c_mine, c_other, wuk_m, wuk_o, wuv_m, wuv_o)


def kernel(x, Wdkv, Wuk, Wuv, Wq, Wqr, Wkr, Wo):
    x2 = x.reshape(B * S, D)
    Q, Qr, Kr, c_mine, c_other, wuk_o, wuv_o = _proj_exchange(
        x2, Wdkv, Wuk, Wuv, Wq, Wqr, Wkr)
    O = _attention(Q, Qr, Kr, c_mine, Wuk, c_other, wuk_o, Wuv, wuv_o)
    out = _matmul(O, Wo)
    return out.reshape(B, S, D)
